# Initial kernel scaffold; baseline (speedup 1.0000x reference)
import jax
import jax.numpy as jnp
from jax.experimental import pallas as pl


def kernel(x):
    raise NotImplementedError("write your pallas kernel here")


# speedup vs baseline: 1.9892x; 1.9892x over previous
import jax
import jax.numpy as jnp
import numpy as np
from jax.experimental import pallas as pl
from jax.experimental.pallas import tpu as pltpu

_ROWS = 32
_N = 4096
_I32_MIN = np.int32(-2147483648)
_I32_MAX = np.int32(2147483647)


def _select_body(mt_ref, c_ref, th_ref, cv_ref):
    mt = mt_ref[...]
    c = c_ref[...]

    valid = mt != jnp.inf

    s = jax.lax.bitcast_convert_type(mt, jnp.int32)
    key = jnp.where(s >= 0, s, s ^ _I32_MAX)
    key = jnp.where(key == jnp.int32(-1), jnp.int32(0), key)
    key = jnp.where(mt != mt, _I32_MAX, key)

    n_valid = jnp.sum(valid.astype(jnp.int32), axis=1, keepdims=True)
    k = (n_valid - 1) // 2

    lo = jnp.full(k.shape, _I32_MIN, jnp.int32)
    for bit in range(31, -1, -1):
        if bit == 31:
            mid = jnp.zeros(k.shape, jnp.int32)
        else:
            mid = lo | jnp.int32(1 << bit)
        cnt = jnp.sum((key < mid).astype(jnp.int32), axis=1, keepdims=True)
        lo = jnp.where(cnt <= k, mid, lo)

    cnt_less = jnp.sum((key < lo).astype(jnp.int32), axis=1, keepdims=True)
    j = k - cnt_less
    eq = key == lo
    iota = jax.lax.broadcasted_iota(jnp.int32, key.shape, 1)
    loi = jnp.zeros(k.shape, jnp.int32)
    for bit in range(11, -1, -1):
        mid = loi | jnp.int32(1 << bit)
        cnt = jnp.sum((eq & (iota < mid)).astype(jnp.int32), axis=1,
                      keepdims=True)
        loi = jnp.where(cnt <= j, mid, loi)

    sel = iota == loi
    th_ref[...] = jnp.sum(jnp.where(sel, mt, 0.0), axis=1, keepdims=True)
    cv_ref[...] = jnp.sum(jnp.where(sel, c, 0.0), axis=1, keepdims=True)


def _median_select(mt, c, interpret=False):
    b = mt.shape[0]
    spec = pl.BlockSpec((_ROWS, _N), lambda i: (i, 0))
    out_spec = pl.BlockSpec((_ROWS, 1), lambda i: (i, 0))
    return pl.pallas_call(
        _select_body,
        grid=(b // _ROWS,),
        in_specs=[spec, spec],
        out_specs=[out_spec, out_spec],
        out_shape=[
            jax.ShapeDtypeStruct((b, 1), jnp.float32),
            jax.ShapeDtypeStruct((b, 1), jnp.float32),
        ],
        compiler_params=pltpu.CompilerParams(
            dimension_semantics=("parallel",),
        ),
        interpret=interpret,
    )(mt, c)


def kernel(x, interpret=False):
    validity = jnp.any(x != 0.0, axis=2)

    y2 = x[:, :, 0:1]
    y1 = x[:, :, 1:2]
    x2 = x[:, :, 2:3]
    x1 = x[:, :, 3:4]

    r1 = jnp.sqrt(x1 ** 2 + y1 ** 2)
    r2 = jnp.sqrt(x2 ** 2 + y2 ** 2)
    a1 = jnp.arctan2(y1, x1)
    a2 = jnp.arctan2(y2, x2)

    stationary = (r1 == r2) & (a1 == a2)
    thetas = 2.0 * jnp.arctan(
        (-jnp.sin(a2) + (r1 / r2) * jnp.sin(a1))
        / ((r1 / r2) * jnp.cos(a1) + jnp.cos(a2))
    )
    radii = r2 * jnp.sin(a1 - a2 - thetas) / (
        2.0 * jnp.sin(thetas / 2.0) * jnp.sin(-a1 + thetas / 2.0)
    )
    radii = jnp.where(stationary, jnp.inf, radii)
    curvatures = 1.0 / radii

    t = thetas[:, :, 0]
    cv = curvatures[:, :, 0]
    mt = jnp.where(validity, t, jnp.inf)

    th_est, cv_est = _median_select(mt, cv, interpret=interpret)
    return jnp.concatenate([th_est, cv_est], axis=1)


# speedup vs baseline: 2.8139x; 1.4146x over previous
import jax
import jax.numpy as jnp
import numpy as np
from jax.experimental import pallas as pl
from jax.experimental.pallas import tpu as pltpu

_ROWS = 32
_N = 4096
_I32_MIN = np.int32(-2147483648)
_I32_MAX = np.int32(2147483647)


def _select_body(mt_ref, c_ref, th_ref, cv_ref):
    mt = mt_ref[...]
    c = c_ref[...]

    valid = mt != jnp.inf

    s = jax.lax.bitcast_convert_type(mt, jnp.int32)
    key = jnp.where(s >= 0, s, s ^ _I32_MAX)
    key = jnp.where(key == jnp.int32(-1), jnp.int32(0), key)
    key = jnp.where(mt != mt, _I32_MAX, key)

    n_valid = jnp.sum(valid.astype(jnp.int32), axis=1, keepdims=True)
    k = (n_valid - 1) // 2

    lo = jnp.full(k.shape, _I32_MIN, jnp.int32)
    for bit in range(31, -1, -1):
        if bit == 31:
            mid = jnp.zeros(k.shape, jnp.int32)
        else:
            mid = lo | jnp.int32(1 << bit)
        cnt = jnp.sum((key < mid).astype(jnp.int32), axis=1, keepdims=True)
        lo = jnp.where(cnt <= k, mid, lo)

    cnt_less = jnp.sum((key < lo).astype(jnp.int32), axis=1, keepdims=True)
    j = k - cnt_less
    eq = key == lo
    iota = jax.lax.broadcasted_iota(jnp.int32, key.shape, 1)
    loi = jnp.zeros(k.shape, jnp.int32)
    for bit in range(11, -1, -1):
        mid = loi | jnp.int32(1 << bit)
        cnt = jnp.sum((eq & (iota < mid)).astype(jnp.int32), axis=1,
                      keepdims=True)
        loi = jnp.where(cnt <= j, mid, loi)

    sel = iota == loi
    th_ref[...] = jnp.sum(jnp.where(sel, mt, 0.0), axis=1, keepdims=True)
    cv_ref[...] = jnp.sum(jnp.where(sel, c, 0.0), axis=1, keepdims=True)


def _median_select(mt, c, interpret=False):
    b = mt.shape[0]
    spec = pl.BlockSpec((_ROWS, _N), lambda i: (i, 0))
    out_spec = pl.BlockSpec((_ROWS, 1), lambda i: (i, 0))
    return pl.pallas_call(
        _select_body,
        grid=(b // _ROWS,),
        in_specs=[spec, spec],
        out_specs=[out_spec, out_spec],
        out_shape=[
            jax.ShapeDtypeStruct((b, 1), jnp.float32),
            jax.ShapeDtypeStruct((b, 1), jnp.float32),
        ],
        compiler_params=pltpu.CompilerParams(
            dimension_semantics=("parallel",),
        ),
        interpret=interpret,
    )(mt, c)


def kernel(x, interpret=False):
    xt = jnp.transpose(x, (2, 0, 1))
    y2 = xt[0]
    y1 = xt[1]
    x2 = xt[2]
    x1 = xt[3]
    validity = (y2 != 0.0) | (y1 != 0.0) | (x2 != 0.0) | (x1 != 0.0)

    r1 = jnp.sqrt(x1 ** 2 + y1 ** 2)
    r2 = jnp.sqrt(x2 ** 2 + y2 ** 2)
    a1 = jnp.arctan2(y1, x1)
    a2 = jnp.arctan2(y2, x2)

    stationary = (r1 == r2) & (a1 == a2)
    thetas = 2.0 * jnp.arctan(
        (-jnp.sin(a2) + (r1 / r2) * jnp.sin(a1))
        / ((r1 / r2) * jnp.cos(a1) + jnp.cos(a2))
    )
    radii = r2 * jnp.sin(a1 - a2 - thetas) / (
        2.0 * jnp.sin(thetas / 2.0) * jnp.sin(-a1 + thetas / 2.0)
    )
    radii = jnp.where(stationary, jnp.inf, radii)
    curvatures = 1.0 / radii

    mt = jnp.where(validity, thetas, jnp.inf)
    cv = curvatures

    th_est, cv_est = _median_select(mt, cv, interpret=interpret)
    return jnp.concatenate([th_est, cv_est], axis=1)


# speedup vs baseline: 3.3013x; 1.1732x over previous
import jax
import jax.numpy as jnp
import numpy as np
from jax.experimental import pallas as pl
from jax.experimental.pallas import tpu as pltpu

_ROWS = 32
_N = 4096
_I32_MIN = np.int32(-2147483648)
_I32_MAX = np.int32(2147483647)


def _select_body(mt_ref, th_ref, idx_ref):
    mt = mt_ref[...]

    valid = mt != jnp.inf

    s = jax.lax.bitcast_convert_type(mt, jnp.int32)
    key = jnp.where(s >= 0, s, s ^ _I32_MAX)
    key = jnp.where(key == jnp.int32(-1), jnp.int32(0), key)
    key = jnp.where(mt != mt, _I32_MAX, key)

    n_valid = jnp.sum(valid.astype(jnp.int32), axis=1, keepdims=True)
    k = (n_valid - 1) // 2

    lo = jnp.full(k.shape, _I32_MIN, jnp.int32)
    for bit in range(31, -1, -1):
        if bit == 31:
            mid = jnp.zeros(k.shape, jnp.int32)
        else:
            mid = lo | jnp.int32(1 << bit)
        cnt = jnp.sum((key < mid).astype(jnp.int32), axis=1, keepdims=True)
        lo = jnp.where(cnt <= k, mid, lo)

    cnt_less = jnp.sum((key < lo).astype(jnp.int32), axis=1, keepdims=True)
    j = k - cnt_less
    eq = key == lo
    iota = jax.lax.broadcasted_iota(jnp.int32, key.shape, 1)
    loi = jnp.zeros(k.shape, jnp.int32)
    for bit in range(11, -1, -1):
        mid = loi | jnp.int32(1 << bit)
        cnt = jnp.sum((eq & (iota < mid)).astype(jnp.int32), axis=1,
                      keepdims=True)
        loi = jnp.where(cnt <= j, mid, loi)

    srec = jnp.where(lo >= 0, lo, lo ^ _I32_MAX)
    th_ref[...] = jax.lax.bitcast_convert_type(srec, jnp.float32)
    idx_ref[...] = loi


def _median_select(mt, interpret=False):
    b = mt.shape[0]
    spec = pl.BlockSpec((_ROWS, _N), lambda i: (i, 0))
    out_spec = pl.BlockSpec((_ROWS, 1), lambda i: (i, 0))
    return pl.pallas_call(
        _select_body,
        grid=(b // _ROWS,),
        in_specs=[spec],
        out_specs=[out_spec, out_spec],
        out_shape=[
            jax.ShapeDtypeStruct((b, 1), jnp.float32),
            jax.ShapeDtypeStruct((b, 1), jnp.int32),
        ],
        compiler_params=pltpu.CompilerParams(
            dimension_semantics=("parallel",),
        ),
        interpret=interpret,
    )(mt)


def _theta_plane(y2, y1, x2, x1):
    r1 = jnp.sqrt(x1 ** 2 + y1 ** 2)
    r2 = jnp.sqrt(x2 ** 2 + y2 ** 2)
    a1 = jnp.arctan2(y1, x1)
    a2 = jnp.arctan2(y2, x2)
    thetas = 2.0 * jnp.arctan(
        (-jnp.sin(a2) + (r1 / r2) * jnp.sin(a1))
        / ((r1 / r2) * jnp.cos(a1) + jnp.cos(a2))
    )
    return thetas


def _curvature_at(y2, y1, x2, x1):
    r1 = jnp.sqrt(x1 ** 2 + y1 ** 2)
    r2 = jnp.sqrt(x2 ** 2 + y2 ** 2)
    a1 = jnp.arctan2(y1, x1)
    a2 = jnp.arctan2(y2, x2)
    thetas = _theta_plane(y2, y1, x2, x1)
    stationary = (r1 == r2) & (a1 == a2)
    radii = r2 * jnp.sin(a1 - a2 - thetas) / (
        2.0 * jnp.sin(thetas / 2.0) * jnp.sin(-a1 + thetas / 2.0)
    )
    radii = jnp.where(stationary, jnp.inf, radii)
    return 1.0 / radii


def kernel(x, interpret=False):
    xt = jnp.transpose(x, (2, 0, 1))
    y2 = xt[0]
    y1 = xt[1]
    x2 = xt[2]
    x1 = xt[3]
    validity = (y2 != 0.0) | (y1 != 0.0) | (x2 != 0.0) | (x1 != 0.0)

    thetas = _theta_plane(y2, y1, x2, x1)
    mt = jnp.where(validity, thetas, jnp.inf)

    th_est, med_idx = _median_select(mt, interpret=interpret)

    gy2 = jnp.take_along_axis(y2, med_idx, axis=1)
    gy1 = jnp.take_along_axis(y1, med_idx, axis=1)
    gx2 = jnp.take_along_axis(x2, med_idx, axis=1)
    gx1 = jnp.take_along_axis(x1, med_idx, axis=1)
    cv_est = _curvature_at(gy2, gy1, gx2, gx1)

    return jnp.concatenate([th_est, cv_est], axis=1)


# speedup vs baseline: 3.4730x; 1.0520x over previous
import jax
import jax.numpy as jnp
import numpy as np
from jax.experimental import pallas as pl
from jax.experimental.pallas import tpu as pltpu

_LANES = 128
_N = 4096
_I32_MIN = np.int32(-2147483648)
_I32_MAX = np.int32(2147483647)


def _select_body(mt_ref, th_ref, idx_ref):
    mt = mt_ref[...]

    valid = mt != jnp.inf

    s = jax.lax.bitcast_convert_type(mt, jnp.int32)
    key = jnp.where(s >= 0, s, s ^ _I32_MAX)
    key = jnp.where(key == jnp.int32(-1), jnp.int32(0), key)
    key = jnp.where(mt != mt, _I32_MAX, key)

    n_valid = jnp.sum(valid.astype(jnp.int32), axis=0, keepdims=True)
    k = (n_valid - 1) // 2

    lo = jnp.full(k.shape, _I32_MIN, jnp.int32)
    for bit in range(31, -1, -1):
        if bit == 31:
            mid = jnp.zeros(k.shape, jnp.int32)
        else:
            mid = lo | jnp.int32(1 << bit)
        cnt = jnp.sum((key < mid).astype(jnp.int32), axis=0, keepdims=True)
        lo = jnp.where(cnt <= k, mid, lo)

    cnt_less = jnp.sum((key < lo).astype(jnp.int32), axis=0, keepdims=True)
    j = k - cnt_less
    eq = key == lo
    iota = jax.lax.broadcasted_iota(jnp.int32, key.shape, 0)
    loi = jnp.zeros(k.shape, jnp.int32)
    for bit in range(11, -1, -1):
        mid = loi | jnp.int32(1 << bit)
        cnt = jnp.sum((eq & (iota < mid)).astype(jnp.int32), axis=0,
                      keepdims=True)
        loi = jnp.where(cnt <= j, mid, loi)

    srec = jnp.where(lo >= 0, lo, lo ^ _I32_MAX)
    th_ref[...] = jax.lax.bitcast_convert_type(srec, jnp.float32)
    idx_ref[...] = loi


def _median_select(mt_t, interpret=False):
    b = mt_t.shape[1]
    spec = pl.BlockSpec((_N, _LANES), lambda i: (0, i))
    out_spec = pl.BlockSpec((1, _LANES), lambda i: (0, i))
    return pl.pallas_call(
        _select_body,
        grid=(b // _LANES,),
        in_specs=[spec],
        out_specs=[out_spec, out_spec],
        out_shape=[
            jax.ShapeDtypeStruct((1, b), jnp.float32),
            jax.ShapeDtypeStruct((1, b), jnp.int32),
        ],
        compiler_params=pltpu.CompilerParams(
            dimension_semantics=("parallel",),
        ),
        interpret=interpret,
    )(mt_t)


def _theta_plane(y2, y1, x2, x1):
    r1 = jnp.sqrt(x1 ** 2 + y1 ** 2)
    r2 = jnp.sqrt(x2 ** 2 + y2 ** 2)
    a1 = jnp.arctan2(y1, x1)
    a2 = jnp.arctan2(y2, x2)
    thetas = 2.0 * jnp.arctan(
        (-jnp.sin(a2) + (r1 / r2) * jnp.sin(a1))
        / ((r1 / r2) * jnp.cos(a1) + jnp.cos(a2))
    )
    return thetas


def _curvature_at(y2, y1, x2, x1):
    r1 = jnp.sqrt(x1 ** 2 + y1 ** 2)
    r2 = jnp.sqrt(x2 ** 2 + y2 ** 2)
    a1 = jnp.arctan2(y1, x1)
    a2 = jnp.arctan2(y2, x2)
    thetas = _theta_plane(y2, y1, x2, x1)
    stationary = (r1 == r2) & (a1 == a2)
    radii = r2 * jnp.sin(a1 - a2 - thetas) / (
        2.0 * jnp.sin(thetas / 2.0) * jnp.sin(-a1 + thetas / 2.0)
    )
    radii = jnp.where(stationary, jnp.inf, radii)
    return 1.0 / radii


def kernel(x, interpret=False):
    b = x.shape[0]
    xt = jnp.transpose(x, (2, 1, 0))
    y2 = xt[0]
    y1 = xt[1]
    x2 = xt[2]
    x1 = xt[3]
    validity = (y2 != 0.0) | (y1 != 0.0) | (x2 != 0.0) | (x1 != 0.0)

    thetas = _theta_plane(y2, y1, x2, x1)
    mt = jnp.where(validity, thetas, jnp.inf)

    th_est, med_idx = _median_select(mt, interpret=interpret)

    gy2 = jnp.take_along_axis(y2, med_idx, axis=0)
    gy1 = jnp.take_along_axis(y1, med_idx, axis=0)
    gx2 = jnp.take_along_axis(x2, med_idx, axis=0)
    gx1 = jnp.take_along_axis(x1, med_idx, axis=0)
    cv_est = _curvature_at(gy2, gy1, gx2, gx1)

    return jnp.concatenate([th_est.reshape(b, 1), cv_est.reshape(b, 1)],
                           axis=1)


# speedup vs baseline: 4.1318x; 1.1897x over previous
import jax
import jax.numpy as jnp
import numpy as np
from jax.experimental import pallas as pl
from jax.experimental.pallas import tpu as pltpu

_LANES = 128
_N = 4096
_I32_MIN = np.int32(-2147483648)
_I32_MAX = np.int32(2147483647)


def _select_body(mt_ref, th_ref, idx_ref):
    mt = mt_ref[...]

    valid = mt != jnp.inf

    s = jax.lax.bitcast_convert_type(mt, jnp.int32)
    key = jnp.where(s >= 0, s, s ^ _I32_MAX)
    key = jnp.where(key == jnp.int32(-1), jnp.int32(0), key)
    key = jnp.where(mt != mt, _I32_MAX, key)

    n_valid = jnp.sum(valid.astype(jnp.int32), axis=0, keepdims=True)
    k = (n_valid - 1) // 2

    lo = jnp.full(k.shape, _I32_MIN, jnp.int32)
    for bit in range(31, -1, -1):
        if bit == 31:
            mid = jnp.zeros(k.shape, jnp.int32)
        else:
            mid = lo | jnp.int32(1 << bit)
        cnt = jnp.sum((key < mid).astype(jnp.int32), axis=0, keepdims=True)
        lo = jnp.where(cnt <= k, mid, lo)

    cnt_less = jnp.sum((key < lo).astype(jnp.int32), axis=0, keepdims=True)
    j = k - cnt_less
    iota = jax.lax.broadcasted_iota(jnp.int32, key.shape, 0)
    eqi = jnp.where(key == lo, iota, jnp.int32(_N))
    loi = jnp.zeros(k.shape, jnp.int32)
    for bit in range(11, -1, -1):
        mid = loi | jnp.int32(1 << bit)
        cnt = jnp.sum((eqi < mid).astype(jnp.int32), axis=0, keepdims=True)
        loi = jnp.where(cnt <= j, mid, loi)

    srec = jnp.where(lo >= 0, lo, lo ^ _I32_MAX)
    th_ref[...] = jax.lax.bitcast_convert_type(srec, jnp.float32)
    idx_ref[...] = loi


def _median_select(mt_t, interpret=False):
    b = mt_t.shape[1]
    spec = pl.BlockSpec((_N, _LANES), lambda i: (0, i))
    out_spec = pl.BlockSpec((1, _LANES), lambda i: (0, i))
    return pl.pallas_call(
        _select_body,
        grid=(b // _LANES,),
        in_specs=[spec],
        out_specs=[out_spec, out_spec],
        out_shape=[
            jax.ShapeDtypeStruct((1, b), jnp.float32),
            jax.ShapeDtypeStruct((1, b), jnp.int32),
        ],
        compiler_params=pltpu.CompilerParams(
            dimension_semantics=("parallel",),
        ),
        interpret=interpret,
    )(mt_t)


def _theta_plane(y2, y1, x2, x1):
    r1 = jnp.sqrt(x1 ** 2 + y1 ** 2)
    r2 = jnp.sqrt(x2 ** 2 + y2 ** 2)
    a1 = jnp.arctan2(y1, x1)
    a2 = jnp.arctan2(y2, x2)
    thetas = 2.0 * jnp.arctan(
        (-jnp.sin(a2) + (r1 / r2) * jnp.sin(a1))
        / ((r1 / r2) * jnp.cos(a1) + jnp.cos(a2))
    )
    return thetas


def _curvature_at(y2, y1, x2, x1):
    r1 = jnp.sqrt(x1 ** 2 + y1 ** 2)
    r2 = jnp.sqrt(x2 ** 2 + y2 ** 2)
    a1 = jnp.arctan2(y1, x1)
    a2 = jnp.arctan2(y2, x2)
    thetas = _theta_plane(y2, y1, x2, x1)
    stationary = (r1 == r2) & (a1 == a2)
    radii = r2 * jnp.sin(a1 - a2 - thetas) / (
        2.0 * jnp.sin(thetas / 2.0) * jnp.sin(-a1 + thetas / 2.0)
    )
    radii = jnp.where(stationary, jnp.inf, radii)
    return 1.0 / radii


def kernel(x, interpret=False):
    b = x.shape[0]
    xt = jnp.transpose(x, (2, 1, 0))
    y2 = xt[0]
    y1 = xt[1]
    x2 = xt[2]
    x1 = xt[3]
    validity = (y2 != 0.0) | (y1 != 0.0) | (x2 != 0.0) | (x1 != 0.0)

    thetas = _theta_plane(y2, y1, x2, x1)
    mt = jnp.where(validity, thetas, jnp.inf)

    th_est, med_idx = _median_select(mt, interpret=interpret)

    g = jnp.take_along_axis(x, med_idx.reshape(b, 1, 1), axis=1)
    cv_est = _curvature_at(g[:, 0, 0], g[:, 0, 1], g[:, 0, 2], g[:, 0, 3])

    return jnp.stack([th_est.reshape(b), cv_est.reshape(b)], axis=1)


# speedup vs baseline: 4.5448x; 1.0999x over previous
import jax
import jax.numpy as jnp
import numpy as np
from jax.experimental import pallas as pl
from jax.experimental.pallas import tpu as pltpu

_LANES = 128
_N = 4096
_I32_MIN = np.int32(-2147483648)
_I32_MAX = np.int32(2147483647)


def _select_body(mt_ref, th_ref, idx_ref):
    mt = mt_ref[...]

    valid = mt != jnp.inf

    s = jax.lax.bitcast_convert_type(mt, jnp.int32)
    key = jnp.where(s >= 0, s, s ^ _I32_MAX)
    key = jnp.where(key == jnp.int32(-1), jnp.int32(0), key)
    key = jnp.where(mt != mt, _I32_MAX, key)

    n_valid = jnp.sum(valid.astype(jnp.int32), axis=0, keepdims=True)
    k = (n_valid - 1) // 2

    lo = jnp.full(k.shape, _I32_MIN, jnp.int32)
    for bit in range(31, -1, -1):
        if bit == 31:
            mid = jnp.zeros(k.shape, jnp.int32)
        else:
            mid = lo | jnp.int32(1 << bit)
        cnt = jnp.sum((key < mid).astype(jnp.int32), axis=0, keepdims=True)
        lo = jnp.where(cnt <= k, mid, lo)

    cnt_less = jnp.sum((key < lo).astype(jnp.int32), axis=0, keepdims=True)
    j = k - cnt_less
    iota = jax.lax.broadcasted_iota(jnp.int32, key.shape, 0)
    eqi = jnp.where(key == lo, iota, jnp.int32(_N))

    def _first_eq(_):
        return jnp.min(eqi, axis=0, keepdims=True)

    def _rank_j(_):
        loi = jnp.zeros(k.shape, jnp.int32)
        for bit in range(11, -1, -1):
            mid = loi | jnp.int32(1 << bit)
            cnt = jnp.sum((eqi < mid).astype(jnp.int32), axis=0,
                          keepdims=True)
            loi = jnp.where(cnt <= j, mid, loi)
        return loi

    loi = jax.lax.cond(jnp.any(j > 0), _rank_j, _first_eq, 0)

    srec = jnp.where(lo >= 0, lo, lo ^ _I32_MAX)
    th_ref[...] = jax.lax.bitcast_convert_type(srec, jnp.float32)
    idx_ref[...] = loi


def _median_select(mt_t, interpret=False):
    b = mt_t.shape[1]
    spec = pl.BlockSpec((_N, _LANES), lambda i: (0, i))
    out_spec = pl.BlockSpec((1, _LANES), lambda i: (0, i))
    return pl.pallas_call(
        _select_body,
        grid=(b // _LANES,),
        in_specs=[spec],
        out_specs=[out_spec, out_spec],
        out_shape=[
            jax.ShapeDtypeStruct((1, b), jnp.float32),
            jax.ShapeDtypeStruct((1, b), jnp.int32),
        ],
        compiler_params=pltpu.CompilerParams(
            dimension_semantics=("parallel",),
        ),
        interpret=interpret,
    )(mt_t)


def _theta_plane(y2, y1, x2, x1):
    r1 = jnp.sqrt(x1 ** 2 + y1 ** 2)
    r2 = jnp.sqrt(x2 ** 2 + y2 ** 2)
    a1 = jnp.arctan2(y1, x1)
    a2 = jnp.arctan2(y2, x2)
    thetas = 2.0 * jnp.arctan(
        (-jnp.sin(a2) + (r1 / r2) * jnp.sin(a1))
        / ((r1 / r2) * jnp.cos(a1) + jnp.cos(a2))
    )
    return thetas


def _curvature_at(y2, y1, x2, x1):
    r1 = jnp.sqrt(x1 ** 2 + y1 ** 2)
    r2 = jnp.sqrt(x2 ** 2 + y2 ** 2)
    a1 = jnp.arctan2(y1, x1)
    a2 = jnp.arctan2(y2, x2)
    thetas = _theta_plane(y2, y1, x2, x1)
    stationary = (r1 == r2) & (a1 == a2)
    radii = r2 * jnp.sin(a1 - a2 - thetas) / (
        2.0 * jnp.sin(thetas / 2.0) * jnp.sin(-a1 + thetas / 2.0)
    )
    radii = jnp.where(stationary, jnp.inf, radii)
    return 1.0 / radii


def kernel(x, interpret=False):
    b = x.shape[0]
    xt = jnp.transpose(x, (2, 1, 0))
    y2 = xt[0]
    y1 = xt[1]
    x2 = xt[2]
    x1 = xt[3]
    validity = (y2 != 0.0) | (y1 != 0.0) | (x2 != 0.0) | (x1 != 0.0)

    thetas = _theta_plane(y2, y1, x2, x1)
    mt = jnp.where(validity, thetas, jnp.inf)

    th_est, med_idx = _median_select(mt, interpret=interpret)

    g = jnp.take_along_axis(x, med_idx.reshape(b, 1, 1), axis=1)
    cv_est = _curvature_at(g[:, 0, 0], g[:, 0, 1], g[:, 0, 2], g[:, 0, 3])

    return jnp.stack([th_est.reshape(b), cv_est.reshape(b)], axis=1)


# speedup vs baseline: 5.3786x; 1.1835x over previous
import jax
import jax.numpy as jnp
import numpy as np
from jax.experimental import pallas as pl
from jax.experimental.pallas import tpu as pltpu

_LANES = 256
_N = 4096
_I32_MIN = np.int32(-2147483648)
_I32_MAX = np.int32(2147483647)


def _select_body(mt_ref, th_ref, idx_ref):
    mt = mt_ref[...]

    valid = mt != jnp.inf

    s = jax.lax.bitcast_convert_type(mt, jnp.int32)
    key = jnp.where(s >= 0, s, s ^ _I32_MAX)
    key = jnp.where(key == jnp.int32(-1), jnp.int32(0), key)
    key = jnp.where(mt != mt, _I32_MAX, key)

    n_valid = jnp.sum(valid.astype(jnp.int32), axis=0, keepdims=True)
    k = (n_valid - 1) // 2

    lo = jnp.full(k.shape, _I32_MIN, jnp.int32)
    for bit in range(31, -1, -1):
        if bit == 31:
            mid = jnp.zeros(k.shape, jnp.int32)
        else:
            mid = lo | jnp.int32(1 << bit)
        cnt = jnp.sum((key < mid).astype(jnp.int32), axis=0, keepdims=True)
        lo = jnp.where(cnt <= k, mid, lo)

    cnt_less = jnp.sum((key < lo).astype(jnp.int32), axis=0, keepdims=True)
    j = k - cnt_less
    iota = jax.lax.broadcasted_iota(jnp.int32, key.shape, 0)
    eqi = jnp.where(key == lo, iota, jnp.int32(_N))

    def _first_eq(_):
        return jnp.min(eqi, axis=0, keepdims=True)

    def _rank_j(_):
        loi = jnp.zeros(k.shape, jnp.int32)
        for bit in range(11, -1, -1):
            mid = loi | jnp.int32(1 << bit)
            cnt = jnp.sum((eqi < mid).astype(jnp.int32), axis=0,
                          keepdims=True)
            loi = jnp.where(cnt <= j, mid, loi)
        return loi

    loi = jax.lax.cond(jnp.any(j > 0), _rank_j, _first_eq, 0)

    srec = jnp.where(lo >= 0, lo, lo ^ _I32_MAX)
    th_ref[...] = jax.lax.bitcast_convert_type(srec, jnp.float32)
    idx_ref[...] = loi


def _median_select(mt_t, interpret=False):
    b = mt_t.shape[1]
    spec = pl.BlockSpec((_N, _LANES), lambda i: (0, i))
    out_spec = pl.BlockSpec((1, _LANES), lambda i: (0, i))
    return pl.pallas_call(
        _select_body,
        grid=(b // _LANES,),
        in_specs=[spec],
        out_specs=[out_spec, out_spec],
        out_shape=[
            jax.ShapeDtypeStruct((1, b), jnp.float32),
            jax.ShapeDtypeStruct((1, b), jnp.int32),
        ],
        compiler_params=pltpu.CompilerParams(
            dimension_semantics=("parallel",),
        ),
        interpret=interpret,
    )(mt_t)


def _theta_plane(y2, y1, x2, x1):
    r1 = jnp.sqrt(x1 ** 2 + y1 ** 2)
    r2 = jnp.sqrt(x2 ** 2 + y2 ** 2)
    a1 = jnp.arctan2(y1, x1)
    a2 = jnp.arctan2(y2, x2)
    thetas = 2.0 * jnp.arctan(
        (-jnp.sin(a2) + (r1 / r2) * jnp.sin(a1))
        / ((r1 / r2) * jnp.cos(a1) + jnp.cos(a2))
    )
    return thetas


def _curvature_at(y2, y1, x2, x1):
    r1 = jnp.sqrt(x1 ** 2 + y1 ** 2)
    r2 = jnp.sqrt(x2 ** 2 + y2 ** 2)
    a1 = jnp.arctan2(y1, x1)
    a2 = jnp.arctan2(y2, x2)
    thetas = _theta_plane(y2, y1, x2, x1)
    stationary = (r1 == r2) & (a1 == a2)
    radii = r2 * jnp.sin(a1 - a2 - thetas) / (
        2.0 * jnp.sin(thetas / 2.0) * jnp.sin(-a1 + thetas / 2.0)
    )
    radii = jnp.where(stationary, jnp.inf, radii)
    return 1.0 / radii


def kernel(x, interpret=False):
    b = x.shape[0]
    xt = jnp.transpose(x, (2, 1, 0))
    y2 = xt[0]
    y1 = xt[1]
    x2 = xt[2]
    x1 = xt[3]
    validity = (y2 != 0.0) | (y1 != 0.0) | (x2 != 0.0) | (x1 != 0.0)

    thetas = _theta_plane(y2, y1, x2, x1)
    mt = jnp.where(validity, thetas, jnp.inf)

    th_est, med_idx = _median_select(mt, interpret=interpret)

    g = jnp.take_along_axis(x, med_idx.reshape(b, 1, 1), axis=1)
    cv_est = _curvature_at(g[:, 0, 0], g[:, 0, 1], g[:, 0, 2], g[:, 0, 3])

    return jnp.stack([th_est.reshape(b), cv_est.reshape(b)], axis=1)
